# Initial kernel scaffold; baseline (speedup 1.0000x reference)
import jax
import jax.numpy as jnp
from jax.experimental import pallas as pl


def kernel(x, edge_index):
    raise NotImplementedError("write your pallas kernel here")


# speedup vs baseline: 3.1191x; 3.1191x over previous
import functools

import jax
import jax.numpy as jnp
from jax import lax
from jax.experimental import pallas as pl
from jax.experimental.pallas import tpu as pltpu
from jax.experimental.pallas import tpu_sc as plsc

_PERM_DNUMS = lax.GatherDimensionNumbers(
    offset_dims=(), collapsed_slice_dims=(0,), start_index_map=(0,))


def _permute(v, idx):
    return lax.gather(v, idx[:, None], _PERM_DNUMS, (1,),
                      mode=lax.GatherScatterMode.PROMISE_IN_BOUNDS)


N_NODES = 10000
D = 128
E = 320000
B = 128
NUM_CHUNKS = E // B
L = 16


def _sc_kernel(x_hbm, ei_hbm, out_hbm,
               idxr_v, idxc_v, zr_v, zc_v, out_v, sem):
    nc = 2
    wid = lax.axis_index("s") * nc + lax.axis_index("c")
    nw = 32

    def chunk_body(i, _):
        c = wid + i * nw
        base = c * B
        pltpu.sync_copy(ei_hbm.at[0, pl.ds(base, B)], idxr_v)
        pltpu.sync_copy(ei_hbm.at[1, pl.ds(base, B)], idxc_v)
        cp1 = pltpu.async_copy(x_hbm.at[idxr_v], zr_v, sem)
        cp2 = pltpu.async_copy(x_hbm.at[idxc_v], zc_v, sem)
        cp1.wait()
        cp2.wait()

        lane = jnp.arange(L, dtype=jnp.int32)
        perms = [lane ^ t for t in (8, 4, 2, 1)]

        def group_body(g, _):
            res = jnp.zeros((L,), jnp.float32)
            for j in range(L):
                e = g * L + j
                acc = zr_v[e, pl.ds(0, L)] * zc_v[e, pl.ds(0, L)]
                for k in range(1, D // L):
                    acc += zr_v[e, pl.ds(k * L, L)] * zc_v[e, pl.ds(k * L, L)]
                for p in perms:
                    acc = acc + _permute(acc, p)
                res = jnp.where(lane == j, acc, res)
            out_v[pl.ds(g * L, L)] = res
            return 0

        lax.fori_loop(0, B // L, group_body, 0)
        pltpu.sync_copy(out_v, out_hbm.at[pl.ds(base, B)])
        return 0

    n_chunks_w = (NUM_CHUNKS - wid + nw - 1) // nw
    lax.fori_loop(0, n_chunks_w, chunk_body, 0)


def kernel(x, edge_index):
    mesh = plsc.VectorSubcoreMesh(core_axis_name="c", subcore_axis_name="s")
    f = functools.partial(
        pl.kernel,
        mesh=mesh,
        out_type=jax.ShapeDtypeStruct((E,), jnp.float32),
        scratch_types=[
            pltpu.VMEM((B,), jnp.int32),
            pltpu.VMEM((B,), jnp.int32),
            pltpu.VMEM((B, D), jnp.float32),
            pltpu.VMEM((B, D), jnp.float32),
            pltpu.VMEM((B,), jnp.float32),
            pltpu.SemaphoreType.DMA,
        ],
    )(_sc_kernel)
    return f(x, edge_index)


# speedup vs baseline: 5.4733x; 1.7547x over previous
import functools

import jax
import jax.numpy as jnp
from jax import lax
from jax.experimental import pallas as pl
from jax.experimental.pallas import tpu as pltpu
from jax.experimental.pallas import tpu_sc as plsc

_PERM_DNUMS = lax.GatherDimensionNumbers(
    offset_dims=(), collapsed_slice_dims=(0,), start_index_map=(0,))


def _permute(v, idx):
    return lax.gather(v, idx[:, None], _PERM_DNUMS, (1,),
                      mode=lax.GatherScatterMode.PROMISE_IN_BOUNDS)


N_NODES = 10000
D = 128
E = 320000
NW = 32
PER_W = E // NW
B = 128
NFULL = PER_W // B
NCHUNK = NFULL + 1
TAIL_BASE = PER_W - B
L = 16


def _sc_kernel(x_hbm, row_hbm, col_hbm, out_hbm,
               idxr_a, idxc_a, zr0, zc0, zr1, zc1, out_a, sem0, sem1):
    nc = 2
    wid = lax.axis_index("s") * nc + lax.axis_index("c")
    base_w = wid * PER_W

    pltpu.sync_copy(row_hbm.at[pl.ds(base_w, PER_W)], idxr_a)
    pltpu.sync_copy(col_hbm.at[pl.ds(base_w, PER_W)], idxc_a)

    bufs = ((zr0, zc0, sem0), (zr1, zc1, sem1))

    def chunk_base(c):
        return jnp.minimum(c * B, TAIL_BASE)

    def issue(c, b):
        base = chunk_base(c)
        zr, zc, sem = bufs[b]
        pltpu.async_copy(x_hbm.at[idxr_a.at[pl.ds(base, B)]], zr, sem)
        pltpu.async_copy(x_hbm.at[idxc_a.at[pl.ds(base, B)]], zc, sem)

    def wait(c, b):
        base = chunk_base(c)
        zr, zc, sem = bufs[b]
        pltpu.make_async_copy(x_hbm.at[idxr_a.at[pl.ds(base, B)]], zr, sem).wait()
        pltpu.make_async_copy(x_hbm.at[idxc_a.at[pl.ds(base, B)]], zc, sem).wait()

    lane = jnp.arange(L, dtype=jnp.int32)
    perms = [lane ^ t for t in (8, 4, 2, 1)]

    def compute(c, b):
        base = chunk_base(c)
        zr, zc, _ = bufs[b]

        def group_body(g, _):
            res = jnp.zeros((L,), jnp.float32)
            for j in range(L):
                e = g * L + j
                acc = zr[e, pl.ds(0, L)] * zc[e, pl.ds(0, L)]
                for k in range(1, D // L):
                    acc += zr[e, pl.ds(k * L, L)] * zc[e, pl.ds(k * L, L)]
                for p in perms:
                    acc = acc + _permute(acc, p)
                res = jnp.where(lane == j, acc, res)
            out_a[pl.ds(base + g * L, L)] = res
            return 0

        lax.fori_loop(0, B // L, group_body, 0)

    issue(0, 0)

    def pair_body(i2, _):
        for b in range(2):
            c = i2 * 2 + b
            issue(c + 1, 1 - b)
            wait(c, b)
            compute(c, b)
        return 0

    lax.fori_loop(0, NFULL // 2, pair_body, 0)
    wait(NCHUNK - 1, 0)
    compute(NCHUNK - 1, 0)

    pltpu.sync_copy(out_a, out_hbm.at[pl.ds(base_w, PER_W)])


def kernel(x, edge_index):
    mesh = plsc.VectorSubcoreMesh(core_axis_name="c", subcore_axis_name="s")
    f = functools.partial(
        pl.kernel,
        mesh=mesh,
        out_type=jax.ShapeDtypeStruct((E,), jnp.float32),
        scratch_types=[
            pltpu.VMEM((PER_W,), jnp.int32),
            pltpu.VMEM((PER_W,), jnp.int32),
            pltpu.VMEM((B, D), jnp.float32),
            pltpu.VMEM((B, D), jnp.float32),
            pltpu.VMEM((B, D), jnp.float32),
            pltpu.VMEM((B, D), jnp.float32),
            pltpu.VMEM((PER_W,), jnp.float32),
            pltpu.SemaphoreType.DMA,
            pltpu.SemaphoreType.DMA,
        ],
    )(_sc_kernel)
    return f(x, edge_index[0], edge_index[1])


# speedup vs baseline: 9.7661x; 1.7843x over previous
import functools

import jax
import jax.numpy as jnp
from jax import lax
from jax.experimental import pallas as pl
from jax.experimental.pallas import tpu as pltpu
from jax.experimental.pallas import tpu_sc as plsc

_PERM_DNUMS = lax.GatherDimensionNumbers(
    offset_dims=(), collapsed_slice_dims=(0,), start_index_map=(0,))


def _permute(v, idx):
    return lax.gather(v, idx[:, None], _PERM_DNUMS, (1,),
                      mode=lax.GatherScatterMode.PROMISE_IN_BOUNDS)


N_NODES = 10000
D = 128
E = 320000
NW = 32
PER_W = E // NW
B = 128
NFULL = PER_W // B
NCHUNK = NFULL + 1
TAIL_BASE = PER_W - B
L = 16


def _sc_kernel(x_hbm, row_hbm, col_hbm, out_hbm,
               idxr_a, idxc_a, zr0, zc0, zr1, zc1, out_a, sem0, sem1):
    nc = 2
    wid = lax.axis_index("s") * nc + lax.axis_index("c")
    base_w = wid * PER_W

    pltpu.sync_copy(row_hbm.at[pl.ds(base_w, PER_W)], idxr_a)
    pltpu.sync_copy(col_hbm.at[pl.ds(base_w, PER_W)], idxc_a)

    bufs = ((zr0, zc0, sem0), (zr1, zc1, sem1))

    def chunk_base(c):
        return jnp.minimum(c * B, TAIL_BASE)

    def issue(c, b):
        base = chunk_base(c)
        zr, zc, sem = bufs[b]
        pltpu.async_copy(x_hbm.at[idxr_a.at[pl.ds(base, B)]], zr, sem)
        pltpu.async_copy(x_hbm.at[idxc_a.at[pl.ds(base, B)]], zc, sem)

    def wait(c, b):
        base = chunk_base(c)
        zr, zc, sem = bufs[b]
        pltpu.make_async_copy(x_hbm.at[idxr_a.at[pl.ds(base, B)]], zr, sem).wait()
        pltpu.make_async_copy(x_hbm.at[idxc_a.at[pl.ds(base, B)]], zc, sem).wait()

    lane = jnp.arange(L, dtype=jnp.int32)
    perms = [lane ^ t for t in (8, 4, 2, 1)]

    def compute(c, b):
        base = chunk_base(c)
        zr, zc, _ = bufs[b]

        def group_body(g, _):
            def edge_body(j, res):
                e = g * L + j
                acc = zr[e, pl.ds(0, L)] * zc[e, pl.ds(0, L)]
                for k in range(1, D // L):
                    acc += zr[e, pl.ds(k * L, L)] * zc[e, pl.ds(k * L, L)]
                for p in perms:
                    acc = acc + _permute(acc, p)
                return jnp.where(lane == j, acc, res)

            res = lax.fori_loop(0, L, edge_body,
                                jnp.zeros((L,), jnp.float32), unroll=4)
            out_a[pl.ds(base + g * L, L)] = res
            return 0

        lax.fori_loop(0, B // L, group_body, 0)

    issue(0, 0)

    def pair_body(i2, _):
        for b in range(2):
            c = i2 * 2 + b
            issue(c + 1, 1 - b)
            wait(c, b)
            compute(c, b)
        return 0

    lax.fori_loop(0, NFULL // 2, pair_body, 0)
    wait(NCHUNK - 1, 0)
    compute(NCHUNK - 1, 0)

    pltpu.sync_copy(out_a, out_hbm.at[pl.ds(base_w, PER_W)])


def kernel(x, edge_index):
    mesh = plsc.VectorSubcoreMesh(core_axis_name="c", subcore_axis_name="s")
    f = functools.partial(
        pl.kernel,
        mesh=mesh,
        out_type=jax.ShapeDtypeStruct((E,), jnp.float32),
        scratch_types=[
            pltpu.VMEM((PER_W,), jnp.int32),
            pltpu.VMEM((PER_W,), jnp.int32),
            pltpu.VMEM((B, D), jnp.float32),
            pltpu.VMEM((B, D), jnp.float32),
            pltpu.VMEM((B, D), jnp.float32),
            pltpu.VMEM((B, D), jnp.float32),
            pltpu.VMEM((PER_W,), jnp.float32),
            pltpu.SemaphoreType.DMA,
            pltpu.SemaphoreType.DMA,
        ],
    )(_sc_kernel)
    return f(x, edge_index[0], edge_index[1])
